# baseline (speedup 1.0000x reference)
import jax
import jax.numpy as jnp
from jax import lax
from jax.experimental import pallas as pl
from jax.experimental.pallas import tpu as pltpu
from jax.experimental.pallas import tpu_sc as plsc

TOPK = 8
NUM_EXPERTS = 64
NW = 32
TPW = 1024
GROUPS = TPW // 16


def _logits_kernel(x_ref, w_ref, lt_ref):
    lt = jax.lax.dot_general(
        w_ref[...], x_ref[...],
        dimension_numbers=(((0,), (1,)), ((), ())),
        preferred_element_type=jnp.float32)
    lt_ref[...] = lt[None]


def _sc_gating(lt_hbm, g_hbm, cnt_hbm, slab, cnt_v):
    w = lax.axis_index("s") * 2 + lax.axis_index("c")
    pltpu.sync_copy(lt_hbm.at[w], slab)

    zero16 = jnp.zeros((16,), jnp.float32)
    for e in range(NUM_EXPERTS):
        cnt_v[e, :] = zero16

    neg_inf = jnp.float32(-jnp.inf)

    def group(t, carry):
        base = t * 16
        p = [slab[e, pl.ds(base, 16)] for e in range(NUM_EXPERTS)]
        m = p[0]
        for e in range(1, NUM_EXPERTS):
            m = jnp.maximum(m, p[e])
        ex = [jnp.exp(p[e] - m) for e in range(NUM_EXPERTS)]
        s = ex[0]
        for e in range(1, NUM_EXPERTS):
            s = s + ex[e]
        r = jnp.float32(1.0) / s
        p = [ex[e] * r for e in range(NUM_EXPERTS)]
        pmax = r
        for e in range(NUM_EXPERTS):
            slab[e, pl.ds(base, 16)] = p[e]
        vals = list(p)
        for _ in range(TOPK):
            mv = vals[0]
            for e in range(1, NUM_EXPERTS):
                mv = jnp.maximum(mv, vals[e])
            for e in range(NUM_EXPERTS):
                vals[e] = jnp.where(vals[e] == mv, neg_inf, vals[e])
        one16 = jnp.ones((16,), jnp.float32)
        z16 = jnp.zeros((16,), jnp.float32)
        e2 = []
        s2 = None
        for e in range(NUM_EXPERTS):
            sel = vals[e] < 0.0
            pe = slab[e, pl.ds(base, 16)]
            v = jnp.where(sel, jnp.exp(pe - pmax), z16)
            e2.append(v)
            s2 = v if s2 is None else s2 + v
            cnt_v[e, :] = cnt_v[e, :] + jnp.where(sel, one16, z16)
        r2 = jnp.float32(1.0) / s2
        for e in range(NUM_EXPERTS):
            slab[e, pl.ds(base, 16)] = e2[e] * r2
        return carry

    lax.fori_loop(0, GROUPS, group, 0)

    pltpu.sync_copy(slab, g_hbm.at[w])
    pltpu.sync_copy(cnt_v, cnt_hbm.at[w])


def kernel(x, w_gate, train):
    del train
    tokens, d = x.shape
    lt3 = pl.pallas_call(
        _logits_kernel,
        grid=(NW,),
        in_specs=[
            pl.BlockSpec((TPW, d), lambda i: (i, 0)),
            pl.BlockSpec((d, NUM_EXPERTS), lambda i: (0, 0)),
        ],
        out_specs=pl.BlockSpec((1, NUM_EXPERTS, TPW), lambda i: (i, 0, 0)),
        out_shape=jax.ShapeDtypeStruct((NW, NUM_EXPERTS, TPW), jnp.float32),
        compiler_params=pltpu.CompilerParams(
            dimension_semantics=("parallel",)),
    )(x, w_gate)

    sc = pl.kernel(
        _sc_gating,
        out_type=[
            jax.ShapeDtypeStruct((NW, NUM_EXPERTS, TPW), jnp.float32),
            jax.ShapeDtypeStruct((NW, NUM_EXPERTS, 16), jnp.float32),
        ],
        mesh=plsc.VectorSubcoreMesh(core_axis_name="c", subcore_axis_name="s"),
        scratch_types=[
            pltpu.VMEM((NUM_EXPERTS, TPW), jnp.float32),
            pltpu.VMEM((NUM_EXPERTS, 16), jnp.float32),
        ],
    )
    g3, cnt = sc(lt3)
    gates = g3.transpose(0, 2, 1).reshape(tokens, NUM_EXPERTS)
    load = cnt.sum(axis=(0, 2)).astype(jnp.int32)
    return gates, load


# speedup vs baseline: 2.2552x; 2.2552x over previous
import jax
import jax.numpy as jnp
from jax.experimental import pallas as pl
from jax.experimental.pallas import tpu as pltpu

TOPK = 8
NUM_EXPERTS = 64


def _gating_kernel(x_ref, w_ref, gates_ref, load_ref):
    logits = jax.lax.dot_general(
        w_ref[...], x_ref[...],
        dimension_numbers=(((0,), (1,)), ((), ())),
        preferred_element_type=jnp.float32)
    m = jnp.max(logits, axis=0, keepdims=True)
    e = jnp.exp(logits - m)
    p = e / jnp.sum(e, axis=0, keepdims=True)

    neg_inf = jnp.float32(-jnp.inf)
    vals = p
    pmax = None
    for i in range(TOPK):
        vmax = jnp.max(vals, axis=0, keepdims=True)
        if i == 0:
            pmax = vmax
        vals = jnp.where(vals == vmax, neg_inf, vals)

    sel = vals < 0.0
    e2 = jnp.where(sel, jnp.exp(p - pmax), 0.0)
    gates_t = e2 / jnp.sum(e2, axis=0, keepdims=True)
    gates_ref[...] = gates_t.T

    load_ref[0, ...] = jnp.sum(sel.astype(jnp.int32), axis=1, keepdims=True)


def kernel(x, w_gate, train):
    del train
    tokens, d = x.shape
    block = 4096
    grid = tokens // block
    gates, load_parts = pl.pallas_call(
        _gating_kernel,
        grid=(grid,),
        in_specs=[
            pl.BlockSpec((block, d), lambda i: (i, 0)),
            pl.BlockSpec((d, NUM_EXPERTS), lambda i: (0, 0)),
        ],
        out_specs=[
            pl.BlockSpec((block, NUM_EXPERTS), lambda i: (i, 0)),
            pl.BlockSpec((1, NUM_EXPERTS, 1), lambda i: (i, 0, 0)),
        ],
        out_shape=[
            jax.ShapeDtypeStruct((tokens, NUM_EXPERTS), jnp.float32),
            jax.ShapeDtypeStruct((grid, NUM_EXPERTS, 1), jnp.int32),
        ],
        compiler_params=pltpu.CompilerParams(
            dimension_semantics=("parallel",)),
    )(x, w_gate)
    return gates, load_parts.sum(axis=0).reshape(NUM_EXPERTS)
